# baseline (speedup 1.0000x reference)
import functools
import math

import jax
import jax.numpy as jnp
from jax.experimental import pallas as pl
from jax.experimental.pallas import tpu as pltpu


def _emb_kernel(ids_ref, w_ref, o_ref, wvm, *, tile, n_load, n_tok, vblk, dh,
                scale):
    c = pl.program_id(0)
    t = pl.program_id(1)

    @pl.when(t < n_load)
    def _pack():
        blk = jax.lax.rem(t + c * (n_load // 2), n_load)
        u = jax.lax.bitcast_convert_type(w_ref[...], jnp.uint32)
        lo = u[:, 0:dh] >> 16
        hi = u[:, dh:2 * dh] & jnp.uint32(0xFFFF0000)
        wvm[pl.ds(blk * vblk, vblk)] = (lo | hi).reshape(vblk, 1, dh)

    @pl.when(t >= n_load)
    def _gather():
        g = t - n_load
        base = (c * n_tok + g) * tile
        for mi in range(tile):
            idx = ids_ref[base + mi]
            w32 = wvm[idx, 0].reshape(1, dh)
            pair = pltpu.bitcast(w32, jnp.bfloat16)
            o_ref[pl.ds(2 * mi, 2), :] = pair.astype(jnp.float32) * scale


def kernel(ids, weight):
    V, D = weight.shape
    orig_shape = ids.shape
    flat = ids.reshape(-1).astype(jnp.int32)
    n = flat.shape[0]
    scale = float(math.sqrt(D))
    dh = D // 2

    flat = jnp.clip(flat, 0, V - 1)

    cores = 2
    tile = 512
    while n % (cores * tile) and tile > 8:
        tile //= 2
    n_pad = ((n + cores * tile - 1) // (cores * tile)) * (cores * tile)
    if n_pad != n:
        flat = jnp.concatenate([flat, jnp.zeros((n_pad - n,), jnp.int32)])
    n_tok = n_pad // (cores * tile)

    vblk = 2048
    while V % vblk:
        vblk //= 2
    n_load = V // vblk

    out = pl.pallas_call(
        functools.partial(
            _emb_kernel, tile=tile, n_load=n_load, n_tok=n_tok, vblk=vblk,
            dh=dh, scale=scale),
        out_shape=jax.ShapeDtypeStruct((4 * tile + 2 * n_pad, dh),
                                       jnp.float32),
        grid_spec=pltpu.PrefetchScalarGridSpec(
            num_scalar_prefetch=1,
            grid=(cores, n_load + n_tok),
            in_specs=[
                pl.BlockSpec(
                    (vblk, D),
                    lambda c, t, ids_smem: (
                        jnp.where(t < n_load,
                                  (t + c * (n_load // 2)) % n_load,
                                  (n_load - 1 + c * (n_load // 2)) % n_load),
                        0),
                ),
            ],
            out_specs=pl.BlockSpec(
                (2 * tile, dh),
                lambda c, t, ids_smem: (
                    jnp.where(t < n_load, c,
                              cores + c * n_tok + t - n_load), 0),
            ),
            scratch_shapes=[
                pltpu.VMEM((V, 1, dh), jnp.uint32),
            ],
        ),
        compiler_params=pltpu.CompilerParams(
            dimension_semantics=("parallel", "parallel"),
            vmem_limit_bytes=60 * 1024 * 1024,
        ),
    )(flat, weight)
    return out[4 * tile: 4 * tile + 2 * n].reshape(*orig_shape, D)
